# baseline (speedup 1.0000x reference)
import functools
import jax
import jax.numpy as jnp
from jax import lax
from jax.experimental import pallas as pl
from jax.experimental.pallas import tpu as pltpu
from jax.experimental.pallas import tpu_sc as plsc

N_BINS = 20
NSEG = N_BINS - 1
_NW = 32
_TOT = 16 * 4 * 512 * 512
_PER_W = _TOT // _NW
_CHUNK = 16384
_NCHUNK = _PER_W // _CHUNK
_GRP = _CHUNK // 16


def _sc_body(x_hbm, rany_hbm, out_hbm, yrow_v, ytab_v, dytab_v,
             inb0_v, inb1_v, outb0_v, outb1_v, sem_in, sem_out):
    inb = (inb0_v, inb1_v)
    outb = (outb0_v, outb1_v)
    cid = lax.axis_index("c")
    sid = lax.axis_index("s")
    wid = sid * 2 + cid
    sample = wid // 2
    base = wid * _PER_W

    pltpu.sync_copy(rany_hbm.at[sample], yrow_v)
    v0 = yrow_v[pl.ds(0, 16)]
    v1 = yrow_v[pl.ds(16, 16)]
    lane = lax.iota(jnp.int32, 16)
    m1 = lane < (N_BINS - 16)
    big = jnp.float32(3.4e38)
    vmin = jnp.minimum(v0, jnp.where(m1, v1, big))
    vmax = jnp.maximum(v0, jnp.where(m1, v1, -big))
    for sh in (8, 4, 2, 1):
        ridx = (lane + sh) & 15
        yrow_v[pl.ds(0, 16)] = vmin
        vmin = jnp.minimum(vmin, plsc.load_gather(yrow_v, [ridx]))
        yrow_v[pl.ds(0, 16)] = vmax
        vmax = jnp.maximum(vmax, plsc.load_gather(yrow_v, [ridx]))
    ymin = vmin
    sc = 1.0 / (vmax - vmin + 1e-5)
    yn0 = (v0 - ymin) * sc
    yn1 = (v1 - ymin) * sc
    ytab_v[pl.ds(0, 16)] = yn0
    ytab_v[pl.ds(16, 16)] = yn1
    dytab_v[pl.ds(0, 16)] = ytab_v[pl.ds(1, 16)] - yn0
    dytab_v[pl.ds(16, 16)] = ytab_v[pl.ds(17, 16)] - ytab_v[pl.ds(16, 16)]

    nineteen = jnp.float32(NSEG)
    maxidx = jnp.int32(NSEG - 1)

    def compute(buf_in, buf_out):
        def body(i, _):
            xv = buf_in[pl.ds(i * 16, 16)]
            t = xv * nineteen
            idx = jnp.minimum(t.astype(jnp.int32), maxidx)
            frac = t - idx.astype(jnp.float32)
            y0 = plsc.load_gather(ytab_v, [idx])
            dy = plsc.load_gather(dytab_v, [idx])
            buf_out[pl.ds(i * 16, 16)] = y0 + dy * frac
            return 0
        lax.fori_loop(0, _GRP, body, 0, unroll=4)

    pltpu.async_copy(x_hbm.at[pl.ds(base, _CHUNK)], inb[0], sem_in).wait()
    for g in range(_NCHUNK):
        cur = g % 2
        nxt = (g + 1) % 2
        if g + 1 < _NCHUNK:
            cp_in = pltpu.async_copy(
                x_hbm.at[pl.ds(base + (g + 1) * _CHUNK, _CHUNK)],
                inb[nxt], sem_in)
        compute(inb[cur], outb[cur])
        if g >= 1:
            pltpu.make_async_copy(
                outb[nxt], out_hbm.at[pl.ds(base + (g - 1) * _CHUNK, _CHUNK)],
                sem_out).wait()
        pltpu.async_copy(
            outb[cur], out_hbm.at[pl.ds(base + g * _CHUNK, _CHUNK)],
            sem_out)
        if g + 1 < _NCHUNK:
            cp_in.wait()
    pltpu.make_async_copy(
        outb[(_NCHUNK - 1) % 2],
        out_hbm.at[pl.ds(base + (_NCHUNK - 1) * _CHUNK, _CHUNK)],
        sem_out).wait()


def _sc_lut(x_flat, ran_y_pad):
    mesh = plsc.VectorSubcoreMesh(core_axis_name="c", subcore_axis_name="s")
    k = functools.partial(
        pl.kernel,
        out_type=jax.ShapeDtypeStruct((_TOT,), jnp.float32),
        mesh=mesh,
        compiler_params=pltpu.CompilerParams(needs_layout_passes=False),
        scratch_types=[
            pltpu.VMEM((32,), jnp.float32),
            pltpu.VMEM((48,), jnp.float32),
            pltpu.VMEM((48,), jnp.float32),
            pltpu.VMEM((_CHUNK,), jnp.float32),
            pltpu.VMEM((_CHUNK,), jnp.float32),
            pltpu.VMEM((_CHUNK,), jnp.float32),
            pltpu.VMEM((_CHUNK,), jnp.float32),
            pltpu.SemaphoreType.DMA,
            pltpu.SemaphoreType.DMA,
        ],
    )(_sc_body)
    return k(x_flat, ran_y_pad)


def kernel(x, ran_y):
    sz = x.shape
    x_flat = x.reshape(-1)
    ran_y_pad = jnp.pad(ran_y, ((0, 0), (0, 32 - N_BINS)))
    out = _sc_lut(x_flat, ran_y_pad)
    return out.reshape(sz)


# speedup vs baseline: 2.9091x; 2.9091x over previous
import functools
import jax
import jax.numpy as jnp
from jax import lax
from jax.experimental import pallas as pl
from jax.experimental.pallas import tpu as pltpu
from jax.experimental.pallas import tpu_sc as plsc

N_BINS = 20
NSEG = N_BINS - 1
_NW = 32
_TOT = 16 * 4 * 512 * 512
_PER_W = _TOT // _NW
_CHUNK = 16384
_NCHUNK = _PER_W // _CHUNK
_GRP = _CHUNK // 16


def _sc_body(x_hbm, rany_hbm, out_hbm, yrow_v, ytab_v, dytab_v,
             inb0_v, inb1_v, outb0_v, outb1_v, sem_in, sem_out):
    inb = (inb0_v, inb1_v)
    outb = (outb0_v, outb1_v)
    cid = lax.axis_index("c")
    sid = lax.axis_index("s")
    wid = sid * 2 + cid
    sample = wid // 2
    base = wid * _PER_W

    pltpu.sync_copy(rany_hbm.at[sample], yrow_v)
    v0 = yrow_v[pl.ds(0, 16)]
    v1 = yrow_v[pl.ds(16, 16)]
    lane = lax.iota(jnp.int32, 16)
    m1 = lane < (N_BINS - 16)
    big = jnp.float32(3.4e38)
    vmin = jnp.minimum(v0, jnp.where(m1, v1, big))
    vmax = jnp.maximum(v0, jnp.where(m1, v1, -big))
    for sh in (8, 4, 2, 1):
        ridx = (lane + sh) & 15
        yrow_v[pl.ds(0, 16)] = vmin
        vmin = jnp.minimum(vmin, plsc.load_gather(yrow_v, [ridx]))
        yrow_v[pl.ds(0, 16)] = vmax
        vmax = jnp.maximum(vmax, plsc.load_gather(yrow_v, [ridx]))
    ymin = vmin
    sc = 1.0 / (vmax - vmin + 1e-5)
    yn0 = (v0 - ymin) * sc
    yn1 = (v1 - ymin) * sc
    ytab_v[pl.ds(0, 16)] = yn0
    ytab_v[pl.ds(16, 16)] = yn1
    dy0 = ytab_v[pl.ds(1, 16)] - yn0
    dy1 = ytab_v[pl.ds(17, 16)] - ytab_v[pl.ds(16, 16)]
    mhi = jnp.int32(-65536)

    def pack(y0v, dyv):
        hb = lax.bitcast_convert_type(y0v, jnp.int32) & mhi
        r = lax.bitcast_convert_type(dyv, jnp.int32)
        r = r + 0x7FFF + ((r >> 16) & 1)
        lb = lax.shift_right_logical(r, 16)
        return hb | lb

    dytab_v[pl.ds(0, 16)] = lax.bitcast_convert_type(pack(yn0, dy0), jnp.float32)
    dytab_v[pl.ds(16, 16)] = lax.bitcast_convert_type(pack(ytab_v[pl.ds(16, 16)], dy1), jnp.float32)

    nineteen = jnp.float32(NSEG)
    maxidx = jnp.int32(NSEG - 1)

    def compute(buf_in, buf_out):
        @plsc.parallel_loop(0, _CHUNK, 16, unroll=8)
        def body(i):
            xv = buf_in[pl.ds(i, 16)]
            t = xv * nineteen
            idx = jnp.minimum(t.astype(jnp.int32), maxidx)
            frac = t - idx.astype(jnp.float32)
            g = lax.bitcast_convert_type(plsc.load_gather(dytab_v, [idx]), jnp.int32)
            y0 = lax.bitcast_convert_type(g & jnp.int32(-65536), jnp.float32)
            dy = lax.bitcast_convert_type(lax.shift_left(g, 16), jnp.float32)
            buf_out[pl.ds(i, 16)] = y0 + dy * frac

    pltpu.async_copy(x_hbm.at[pl.ds(base, _CHUNK)], inb[0], sem_in).wait()
    for g in range(_NCHUNK):
        cur = g % 2
        nxt = (g + 1) % 2
        if g + 1 < _NCHUNK:
            cp_in = pltpu.async_copy(
                x_hbm.at[pl.ds(base + (g + 1) * _CHUNK, _CHUNK)],
                inb[nxt], sem_in)
        compute(inb[cur], outb[cur])
        if g >= 1:
            pltpu.make_async_copy(
                outb[nxt], out_hbm.at[pl.ds(base + (g - 1) * _CHUNK, _CHUNK)],
                sem_out).wait()
        pltpu.async_copy(
            outb[cur], out_hbm.at[pl.ds(base + g * _CHUNK, _CHUNK)],
            sem_out)
        if g + 1 < _NCHUNK:
            cp_in.wait()
    pltpu.make_async_copy(
        outb[(_NCHUNK - 1) % 2],
        out_hbm.at[pl.ds(base + (_NCHUNK - 1) * _CHUNK, _CHUNK)],
        sem_out).wait()


def _sc_lut(x_flat, ran_y_pad):
    mesh = plsc.VectorSubcoreMesh(core_axis_name="c", subcore_axis_name="s")
    k = functools.partial(
        pl.kernel,
        out_type=jax.ShapeDtypeStruct((_TOT,), jnp.float32),
        mesh=mesh,
        compiler_params=pltpu.CompilerParams(needs_layout_passes=False),
        scratch_types=[
            pltpu.VMEM((32,), jnp.float32),
            pltpu.VMEM((48,), jnp.float32),
            pltpu.VMEM((48,), jnp.float32),
            pltpu.VMEM((_CHUNK,), jnp.float32),
            pltpu.VMEM((_CHUNK,), jnp.float32),
            pltpu.VMEM((_CHUNK,), jnp.float32),
            pltpu.VMEM((_CHUNK,), jnp.float32),
            pltpu.SemaphoreType.DMA,
            pltpu.SemaphoreType.DMA,
        ],
    )(_sc_body)
    return k(x_flat, ran_y_pad)


def kernel(x, ran_y):
    sz = x.shape
    x_flat = x.reshape(-1)
    ran_y_pad = jnp.pad(ran_y, ((0, 0), (0, 32 - N_BINS)))
    out = _sc_lut(x_flat, ran_y_pad)
    return out.reshape(sz)


# speedup vs baseline: 2.9382x; 1.0100x over previous
import functools
import jax
import jax.numpy as jnp
from jax import lax
from jax.experimental import pallas as pl
from jax.experimental.pallas import tpu as pltpu
from jax.experimental.pallas import tpu_sc as plsc

N_BINS = 20
NSEG = N_BINS - 1
_NW = 32
_TOT = 16 * 4 * 512 * 512
_PER_W = _TOT // _NW
_CHUNK = 16384
_NCHUNK = _PER_W // _CHUNK
_GRP = _CHUNK // 16


def _sc_body(x_hbm, rany_hbm, out_hbm, yrow_v, ytab_v, dytab_v,
             inb0_v, inb1_v, outb0_v, outb1_v, sem_in, sem_out):
    inb = (inb0_v, inb1_v)
    outb = (outb0_v, outb1_v)
    cid = lax.axis_index("c")
    sid = lax.axis_index("s")
    wid = sid * 2 + cid
    sample = wid // 2
    base = wid * _PER_W

    pltpu.sync_copy(rany_hbm.at[sample], yrow_v)
    v0 = yrow_v[pl.ds(0, 16)]
    v1 = yrow_v[pl.ds(16, 16)]
    lane = lax.iota(jnp.int32, 16)
    m1 = lane < (N_BINS - 16)
    big = jnp.float32(3.4e38)
    vmin = jnp.minimum(v0, jnp.where(m1, v1, big))
    vmax = jnp.maximum(v0, jnp.where(m1, v1, -big))
    for sh in (8, 4, 2, 1):
        ridx = (lane + sh) & 15
        yrow_v[pl.ds(0, 16)] = vmin
        vmin = jnp.minimum(vmin, plsc.load_gather(yrow_v, [ridx]))
        yrow_v[pl.ds(0, 16)] = vmax
        vmax = jnp.maximum(vmax, plsc.load_gather(yrow_v, [ridx]))
    ymin = vmin
    sc = 1.0 / (vmax - vmin + 1e-5)
    yn0 = (v0 - ymin) * sc
    yn1 = (v1 - ymin) * sc
    ytab_v[pl.ds(0, 16)] = yn0
    ytab_v[pl.ds(16, 16)] = yn1
    dy0 = ytab_v[pl.ds(1, 16)] - yn0
    dy1 = ytab_v[pl.ds(17, 16)] - ytab_v[pl.ds(16, 16)]
    mhi = jnp.int32(-65536)

    def rne(v):
        r = lax.bitcast_convert_type(v, jnp.int32)
        return (r + 0x7FFF + ((r >> 16) & 1)) & mhi

    def pack(y0v, dyv):
        return rne(y0v) | lax.shift_right_logical(rne(dyv), 16)

    dytab_v[pl.ds(0, 16)] = lax.bitcast_convert_type(pack(yn0, dy0), jnp.float32)
    dytab_v[pl.ds(16, 16)] = lax.bitcast_convert_type(pack(ytab_v[pl.ds(16, 16)], dy1), jnp.float32)

    nineteen = jnp.float32(NSEG)
    maxidx = jnp.int32(NSEG - 1)

    def compute(buf_in, buf_out):
        @plsc.parallel_loop(0, _CHUNK, 16, unroll=16)
        def body(i):
            xv = buf_in[pl.ds(i, 16)]
            t = xv * nineteen
            idx = jnp.minimum(t.astype(jnp.int32), maxidx)
            frac = t - idx.astype(jnp.float32)
            g = lax.bitcast_convert_type(plsc.load_gather(dytab_v, [idx]), jnp.int32)
            y0 = lax.bitcast_convert_type(g & jnp.int32(-65536), jnp.float32)
            dy = lax.bitcast_convert_type(lax.shift_left(g, 16), jnp.float32)
            buf_out[pl.ds(i, 16)] = y0 + dy * frac

    pltpu.async_copy(x_hbm.at[pl.ds(base, _CHUNK)], inb[0], sem_in).wait()
    for g in range(_NCHUNK):
        cur = g % 2
        nxt = (g + 1) % 2
        if g + 1 < _NCHUNK:
            cp_in = pltpu.async_copy(
                x_hbm.at[pl.ds(base + (g + 1) * _CHUNK, _CHUNK)],
                inb[nxt], sem_in)
        compute(inb[cur], outb[cur])
        if g >= 1:
            pltpu.make_async_copy(
                outb[nxt], out_hbm.at[pl.ds(base + (g - 1) * _CHUNK, _CHUNK)],
                sem_out).wait()
        pltpu.async_copy(
            outb[cur], out_hbm.at[pl.ds(base + g * _CHUNK, _CHUNK)],
            sem_out)
        if g + 1 < _NCHUNK:
            cp_in.wait()
    pltpu.make_async_copy(
        outb[(_NCHUNK - 1) % 2],
        out_hbm.at[pl.ds(base + (_NCHUNK - 1) * _CHUNK, _CHUNK)],
        sem_out).wait()


def _sc_lut(x_flat, ran_y_pad):
    mesh = plsc.VectorSubcoreMesh(core_axis_name="c", subcore_axis_name="s")
    k = functools.partial(
        pl.kernel,
        out_type=jax.ShapeDtypeStruct((_TOT,), jnp.float32),
        mesh=mesh,
        compiler_params=pltpu.CompilerParams(needs_layout_passes=False),
        scratch_types=[
            pltpu.VMEM((32,), jnp.float32),
            pltpu.VMEM((48,), jnp.float32),
            pltpu.VMEM((48,), jnp.float32),
            pltpu.VMEM((_CHUNK,), jnp.float32),
            pltpu.VMEM((_CHUNK,), jnp.float32),
            pltpu.VMEM((_CHUNK,), jnp.float32),
            pltpu.VMEM((_CHUNK,), jnp.float32),
            pltpu.SemaphoreType.DMA,
            pltpu.SemaphoreType.DMA,
        ],
    )(_sc_body)
    return k(x_flat, ran_y_pad)


def kernel(x, ran_y):
    sz = x.shape
    x_flat = x.reshape(-1)
    ran_y_pad = jnp.pad(ran_y, ((0, 0), (0, 32 - N_BINS)))
    out = _sc_lut(x_flat, ran_y_pad)
    return out.reshape(sz)


# speedup vs baseline: 3.1139x; 1.0598x over previous
import functools
import jax
import jax.numpy as jnp
from jax import lax
from jax.experimental import pallas as pl
from jax.experimental.pallas import tpu as pltpu
from jax.experimental.pallas import tpu_sc as plsc

N_BINS = 20
NSEG = N_BINS - 1
_NW = 32
_TOT = 16 * 4 * 512 * 512
_PER_W = _TOT // _NW
_CHUNK = 16384
_NCHUNK = _PER_W // _CHUNK
_GRP = _CHUNK // 16


def _sc_body(x_hbm, rany_hbm, out_hbm, yrow_v, ytab_v, atab_v, btab_v,
             inb0_v, inb1_v, outb0_v, outb1_v, sem_in, sem_out):
    inb = (inb0_v, inb1_v)
    outb = (outb0_v, outb1_v)
    cid = lax.axis_index("c")
    sid = lax.axis_index("s")
    wid = sid * 2 + cid
    sample = wid // 2
    base = wid * _PER_W

    pltpu.sync_copy(rany_hbm.at[sample], yrow_v)
    v0 = yrow_v[pl.ds(0, 16)]
    v1 = yrow_v[pl.ds(16, 16)]
    lane = lax.iota(jnp.int32, 16)
    m1 = lane < (N_BINS - 16)
    big = jnp.float32(3.4e38)
    vmin = jnp.minimum(v0, jnp.where(m1, v1, big))
    vmax = jnp.maximum(v0, jnp.where(m1, v1, -big))
    for sh in (8, 4, 2, 1):
        ridx = (lane + sh) & 15
        yrow_v[pl.ds(0, 16)] = vmin
        vmin = jnp.minimum(vmin, plsc.load_gather(yrow_v, [ridx]))
        yrow_v[pl.ds(0, 16)] = vmax
        vmax = jnp.maximum(vmax, plsc.load_gather(yrow_v, [ridx]))
    ymin = vmin
    sc = 1.0 / (vmax - vmin + 1e-5)
    yn0 = (v0 - ymin) * sc
    yn1 = (v1 - ymin) * sc
    ytab_v[pl.ds(0, 16)] = yn0
    ytab_v[pl.ds(16, 16)] = yn1
    dy0 = ytab_v[pl.ds(1, 16)] - yn0
    dy1 = ytab_v[pl.ds(17, 16)] - ytab_v[pl.ds(16, 16)]
    k0 = lane.astype(jnp.float32)
    k1 = k0 + 16.0
    atab_v[pl.ds(0, 16)] = yn0 - k0 * dy0
    atab_v[pl.ds(16, 16)] = yn1 - k1 * dy1
    btab_v[pl.ds(0, 16)] = dy0
    btab_v[pl.ds(16, 16)] = dy1

    nineteen = jnp.float32(NSEG)

    def compute(buf_in, buf_out):
        @plsc.parallel_loop(0, _CHUNK, 16, unroll=16)
        def body(i):
            xv = buf_in[pl.ds(i, 16)]
            t = xv * nineteen
            idx = t.astype(jnp.int32)
            a = plsc.load_gather(atab_v, [idx])
            b = plsc.load_gather(btab_v, [idx])
            buf_out[pl.ds(i, 16)] = a + b * t

    pltpu.async_copy(x_hbm.at[pl.ds(base, _CHUNK)], inb[0], sem_in).wait()
    for g in range(_NCHUNK):
        cur = g % 2
        nxt = (g + 1) % 2
        if g + 1 < _NCHUNK:
            cp_in = pltpu.async_copy(
                x_hbm.at[pl.ds(base + (g + 1) * _CHUNK, _CHUNK)],
                inb[nxt], sem_in)
        compute(inb[cur], outb[cur])
        if g >= 1:
            pltpu.make_async_copy(
                outb[nxt], out_hbm.at[pl.ds(base + (g - 1) * _CHUNK, _CHUNK)],
                sem_out).wait()
        pltpu.async_copy(
            outb[cur], out_hbm.at[pl.ds(base + g * _CHUNK, _CHUNK)],
            sem_out)
        if g + 1 < _NCHUNK:
            cp_in.wait()
    pltpu.make_async_copy(
        outb[(_NCHUNK - 1) % 2],
        out_hbm.at[pl.ds(base + (_NCHUNK - 1) * _CHUNK, _CHUNK)],
        sem_out).wait()


def _sc_lut(x_flat, ran_y_pad):
    mesh = plsc.VectorSubcoreMesh(core_axis_name="c", subcore_axis_name="s")
    k = functools.partial(
        pl.kernel,
        out_type=jax.ShapeDtypeStruct((_TOT,), jnp.float32),
        mesh=mesh,
        compiler_params=pltpu.CompilerParams(needs_layout_passes=False),
        scratch_types=[
            pltpu.VMEM((32,), jnp.float32),
            pltpu.VMEM((48,), jnp.float32),
            pltpu.VMEM((48,), jnp.float32),
            pltpu.VMEM((48,), jnp.float32),
            pltpu.VMEM((_CHUNK,), jnp.float32),
            pltpu.VMEM((_CHUNK,), jnp.float32),
            pltpu.VMEM((_CHUNK,), jnp.float32),
            pltpu.VMEM((_CHUNK,), jnp.float32),
            pltpu.SemaphoreType.DMA,
            pltpu.SemaphoreType.DMA,
        ],
    )(_sc_body)
    return k(x_flat, ran_y_pad)


def kernel(x, ran_y):
    sz = x.shape
    x_flat = x.reshape(-1)
    ran_y_pad = jnp.pad(ran_y, ((0, 0), (0, 32 - N_BINS)))
    out = _sc_lut(x_flat, ran_y_pad)
    return out.reshape(sz)


# speedup vs baseline: 3.1155x; 1.0005x over previous
import functools
import jax
import jax.numpy as jnp
from jax import lax
from jax.experimental import pallas as pl
from jax.experimental.pallas import tpu as pltpu
from jax.experimental.pallas import tpu_sc as plsc

N_BINS = 20
NSEG = N_BINS - 1
_NW = 32
_TOT = 16 * 4 * 512 * 512
_PER_W = _TOT // _NW
_CHUNK = 16384
_NCHUNK = _PER_W // _CHUNK
_GRP = _CHUNK // 16


def _sc_body(x_hbm, rany_hbm, out_hbm, yrow_v, ytab_v, atab_v, btab_v,
             inb0_v, inb1_v, inb2_v, outb0_v, outb1_v, outb2_v, sem_in, sem_out):
    inb = (inb0_v, inb1_v, inb2_v)
    outb = (outb0_v, outb1_v, outb2_v)
    cid = lax.axis_index("c")
    sid = lax.axis_index("s")
    wid = sid * 2 + cid
    sample = wid // 2
    base = wid * _PER_W

    pltpu.sync_copy(rany_hbm.at[sample], yrow_v)
    v0 = yrow_v[pl.ds(0, 16)]
    v1 = yrow_v[pl.ds(16, 16)]
    lane = lax.iota(jnp.int32, 16)
    m1 = lane < (N_BINS - 16)
    big = jnp.float32(3.4e38)
    vmin = jnp.minimum(v0, jnp.where(m1, v1, big))
    vmax = jnp.maximum(v0, jnp.where(m1, v1, -big))
    for sh in (8, 4, 2, 1):
        ridx = (lane + sh) & 15
        yrow_v[pl.ds(0, 16)] = vmin
        vmin = jnp.minimum(vmin, plsc.load_gather(yrow_v, [ridx]))
        yrow_v[pl.ds(0, 16)] = vmax
        vmax = jnp.maximum(vmax, plsc.load_gather(yrow_v, [ridx]))
    ymin = vmin
    sc = 1.0 / (vmax - vmin + 1e-5)
    yn0 = (v0 - ymin) * sc
    yn1 = (v1 - ymin) * sc
    ytab_v[pl.ds(0, 16)] = yn0
    ytab_v[pl.ds(16, 16)] = yn1
    dy0 = ytab_v[pl.ds(1, 16)] - yn0
    dy1 = ytab_v[pl.ds(17, 16)] - ytab_v[pl.ds(16, 16)]
    k0 = lane.astype(jnp.float32)
    k1 = k0 + 16.0
    atab_v[pl.ds(0, 16)] = yn0 - k0 * dy0
    atab_v[pl.ds(16, 16)] = yn1 - k1 * dy1
    btab_v[pl.ds(0, 16)] = dy0
    btab_v[pl.ds(16, 16)] = dy1

    nineteen = jnp.float32(NSEG)

    def compute(buf_in, buf_out):
        @plsc.parallel_loop(0, _CHUNK, 16, unroll=16)
        def body(i):
            xv = buf_in[pl.ds(i, 16)]
            t = xv * nineteen
            idx = t.astype(jnp.int32)
            a = plsc.load_gather(atab_v, [idx])
            b = plsc.load_gather(btab_v, [idx])
            buf_out[pl.ds(i, 16)] = a + b * t

    _NB = 3
    cps_in = {}
    for g in range(min(_NB - 1, _NCHUNK)):
        cps_in[g] = pltpu.async_copy(
            x_hbm.at[pl.ds(base + g * _CHUNK, _CHUNK)], inb[g % _NB], sem_in)
    for g in range(_NCHUNK):
        if g + _NB - 1 < _NCHUNK:
            gg = g + _NB - 1
            cps_in[gg] = pltpu.async_copy(
                x_hbm.at[pl.ds(base + gg * _CHUNK, _CHUNK)], inb[gg % _NB], sem_in)
        cps_in[g].wait()
        if g >= _NB:
            pltpu.make_async_copy(
                outb[g % _NB],
                out_hbm.at[pl.ds(base + (g - _NB) * _CHUNK, _CHUNK)],
                sem_out).wait()
        compute(inb[g % _NB], outb[g % _NB])
        pltpu.async_copy(
            outb[g % _NB], out_hbm.at[pl.ds(base + g * _CHUNK, _CHUNK)], sem_out)
    for g in range(max(0, _NCHUNK - _NB), _NCHUNK):
        pltpu.make_async_copy(
            outb[g % _NB], out_hbm.at[pl.ds(base + g * _CHUNK, _CHUNK)],
            sem_out).wait()


def _sc_lut(x_flat, ran_y_pad):
    mesh = plsc.VectorSubcoreMesh(core_axis_name="c", subcore_axis_name="s")
    k = functools.partial(
        pl.kernel,
        out_type=jax.ShapeDtypeStruct((_TOT,), jnp.float32),
        mesh=mesh,
        compiler_params=pltpu.CompilerParams(needs_layout_passes=False),
        scratch_types=[
            pltpu.VMEM((32,), jnp.float32),
            pltpu.VMEM((48,), jnp.float32),
            pltpu.VMEM((48,), jnp.float32),
            pltpu.VMEM((48,), jnp.float32),
            pltpu.VMEM((_CHUNK,), jnp.float32),
            pltpu.VMEM((_CHUNK,), jnp.float32),
            pltpu.VMEM((_CHUNK,), jnp.float32),
            pltpu.VMEM((_CHUNK,), jnp.float32),
            pltpu.VMEM((_CHUNK,), jnp.float32),
            pltpu.VMEM((_CHUNK,), jnp.float32),
            pltpu.SemaphoreType.DMA,
            pltpu.SemaphoreType.DMA,
        ],
    )(_sc_body)
    return k(x_flat, ran_y_pad)


def kernel(x, ran_y):
    sz = x.shape
    x_flat = x.reshape(-1)
    ran_y_pad = jnp.pad(ran_y, ((0, 0), (0, 32 - N_BINS)))
    out = _sc_lut(x_flat, ran_y_pad)
    return out.reshape(sz)
